# Initial kernel scaffold; baseline (speedup 1.0000x reference)
import jax
import jax.numpy as jnp
from jax.experimental import pallas as pl


def kernel(centers, sigmas, intensities):
    raise NotImplementedError("write your pallas kernel here")


# speedup vs baseline: 10.9863x; 10.9863x over previous
import functools

import jax
import jax.numpy as jnp
from jax import lax
from jax.experimental import pallas as pl
from jax.experimental.pallas import tpu as pltpu
from jax.experimental.pallas import tpu_sc as plsc

_SHAPE = (96, 96, 96)
_N = 1024
_NCORES = 2
_NSUB = 16
_NW = _NCORES * _NSUB
_PLANES = _SHAPE[0] // _NW
_PLANE_SZ = _SHAPE[1] * _SHAPE[2]
_SLAB_SZ = _PLANES * _PLANE_SZ
_INV_SCALE = 1.0 / 95.0


def _splat_body(pf_hbm, pi_hbm, out_hbm, pf_v, pi_v, vol_v, bvec_v):
    wid = lax.axis_index("c") * _NSUB + lax.axis_index("s")
    x_base = wid * _PLANES

    pltpu.sync_copy(pf_hbm, pf_v)
    pltpu.sync_copy(pi_hbm, pi_v)

    zeros = jnp.zeros((16,), jnp.float32)

    def zero_chunk(i, carry):
        vol_v[pl.ds(i * 16, 16)] = zeros
        return carry

    lax.fori_loop(0, _SLAB_SZ // 16, zero_chunk, 0)

    lane = lax.broadcasted_iota(jnp.int32, (16,), 0)

    def per_gaussian(g, carry):
        irow = pi_v[pl.ds(g * 16, 16)]
        lo0 = irow[0]
        hi0 = irow[1]

        @pl.when((hi0 > x_base) & (lo0 < x_base + _PLANES))
        def _():
            lo1 = irow[2]
            hi1 = irow[3]
            lo2 = irow[4]
            hi2 = irow[5]
            y0 = irow[6]
            z0 = irow[7]
            frow = pf_v[pl.ds(g * 16, 16)]
            cx = frow[0]
            cy = frow[1]
            cz = frow[2]
            inv = frow[3]
            inten = frow[4]

            xi = lane + x_base
            dx = xi.astype(jnp.float32) * _INV_SCALE - cx
            amask = (xi >= lo0) & (xi < hi0) & (lane < _PLANES)
            avec = jnp.where(amask, jnp.exp(-inv * (dx * dx)), 0.0)
            a0 = avec[0]
            a1 = avec[1]
            a2 = avec[2]

            cvecs = []
            for h in range(2):
                zi = lane + (z0 + 16 * h)
                dz = zi.astype(jnp.float32) * _INV_SCALE - cz
                zmask = (zi >= lo2) & (zi < hi2)
                cvecs.append(
                    jnp.where(zmask, inten * jnp.exp(-inv * (dz * dz)), 0.0))
            c0, c1 = cvecs

            for h in range(2):
                yi = lane + (y0 + 16 * h)
                dy = yi.astype(jnp.float32) * _INV_SCALE - cy
                ymask = (yi >= lo1) & (yi < hi1)
                bvec_v[pl.ds(16 * h, 16)] = jnp.where(
                    ymask, jnp.exp(-inv * (dy * dy)), 0.0)

            def y_step(j, yc):
                t = bvec_v[pl.ds(j - y0, 16)][0]
                base = j * _SHAPE[2] + z0
                for px, a in ((0, a0), (1, a1), (2, a2)):
                    coef = a * t
                    off = px * _PLANE_SZ + base
                    r0 = vol_v[pl.ds(off, 16)]
                    vol_v[pl.ds(off, 16)] = r0 + coef * c0
                    r1 = vol_v[pl.ds(off + 16, 16)]
                    vol_v[pl.ds(off + 16, 16)] = r1 + coef * c1
                return yc

            lax.fori_loop(lo1, hi1, y_step, 0)

        return carry

    lax.fori_loop(0, _N, per_gaussian, 0)

    pltpu.sync_copy(vol_v, out_hbm.at[pl.ds(wid * _SLAB_SZ, _SLAB_SZ)])


@functools.partial(jax.jit)
def kernel(centers, sigmas, intensities):
    scale = jnp.float32(_SHAPE[0] - 1)
    c_idx = centers * scale
    cutoff = (3.0 * sigmas * scale)[:, None]
    lo = jnp.maximum(c_idx - cutoff, 0.0).astype(jnp.int32)
    hi = jnp.minimum(jnp.minimum(c_idx + cutoff, scale) + 1.0,
                     jnp.float32(_SHAPE[0])).astype(jnp.int32)
    inv = 0.5 / (sigmas * sigmas)
    zero = jnp.zeros_like(sigmas)
    pf = jnp.stack(
        [centers[:, 0], centers[:, 1], centers[:, 2], inv, intensities,
         zero, zero, zero, zero, zero, zero, zero, zero, zero, zero, zero],
        axis=1).reshape(-1)
    y0 = jnp.minimum(lo[:, 1], _SHAPE[1] - 32)
    z0 = jnp.minimum(lo[:, 2], _SHAPE[2] - 32)
    izero = jnp.zeros_like(y0)
    pi = jnp.stack(
        [lo[:, 0], hi[:, 0], lo[:, 1], hi[:, 1], lo[:, 2], hi[:, 2],
         y0, z0, izero, izero, izero, izero, izero, izero, izero, izero],
        axis=1).reshape(-1)

    mesh = plsc.VectorSubcoreMesh(core_axis_name="c", subcore_axis_name="s",
                                  num_cores=_NCORES, num_subcores=_NSUB)
    splat = pl.kernel(
        _splat_body,
        out_type=jax.ShapeDtypeStruct((_NW * _SLAB_SZ,), jnp.float32),
        mesh=mesh,
        scratch_types=[
            pltpu.VMEM((_N * 16,), jnp.float32),
            pltpu.VMEM((_N * 16,), jnp.int32),
            pltpu.VMEM((_SLAB_SZ,), jnp.float32),
            pltpu.VMEM((48,), jnp.float32),
        ],
    )
    return splat(pf, pi).reshape(_SHAPE)
